# baseline (speedup 1.0000x reference)
import jax
import jax.numpy as jnp
from jax.experimental import pallas as pl
from jax.experimental.pallas import tpu as pltpu

N = 1024
H = 64
BJ = 256
K1 = 8
K2 = 8

_HI = jax.lax.Precision.HIGHEST


def _prep_body(x_ref, wl_ref, wr_ref, attc_ref, we_ref,
               xl_ref, xlg_ref, xrt_ref, al_ref, art_ref, aw_ref):
    x = x_ref[:].astype(jnp.bfloat16)
    wl = wl_ref[:].astype(jnp.bfloat16)
    xl = jnp.dot(x, wl, preferred_element_type=jnp.float32)
    xl_ref[:] = xl
    for k1 in range(K1):
        xlg_ref[k1] = xl[:, k1 * K2:(k1 + 1) * K2]
    xrt = jax.lax.dot_general(wr_ref[:].astype(jnp.bfloat16), x,
                              (((0,), (1,)), ((), ())),
                              preferred_element_type=jnp.float32)
    xrt_ref[:] = xrt
    attc = attc_ref[:]
    al_ref[:] = jnp.dot(xl, attc, precision=_HI,
                        preferred_element_type=jnp.float32)
    art_ref[:] = jax.lax.dot_general(attc, xrt, (((0,), (0,)), ((), ())),
                                     precision=_HI,
                                     preferred_element_type=jnp.float32)
    aw_ref[:] = jnp.dot(we_ref[:], attc, precision=_HI,
                        preferred_element_type=jnp.float32)


def _prep(x, wl, wr, attc, we):
    return pl.pallas_call(
        _prep_body,
        out_shape=(
            jax.ShapeDtypeStruct((N, H), jnp.float32),
            jax.ShapeDtypeStruct((K1, N, K2), jnp.float32),
            jax.ShapeDtypeStruct((H, N), jnp.float32),
            jax.ShapeDtypeStruct((N, 1), jnp.float32),
            jax.ShapeDtypeStruct((1, N), jnp.float32),
            jax.ShapeDtypeStruct((1, 1), jnp.float32),
        ),
    )(x, wl, wr, attc, we)


def _layer_body(thr_ref, aw_ref, att_ref, we_ref,
                ew_ref, xlg_ref, xrt_ref, xl_ref, al_ref, art_ref, b_ref,
                out_ref, acc_ref):
    ew = ew_ref[:]
    acc_ref[:] = 0.6 * (al_ref[:] + art_ref[:] + aw_ref[0, 0] * ew)

    def kgroup(k1, carry):
        xlg = xlg_ref[k1]
        xrt = xrt_ref[pl.ds(k1 * K2, K2), :]
        contrib = None
        for k2 in range(K2):
            w = we_ref[0, k1 * K2 + k2]
            c = 0.4 * att_ref[0, k1 * K2 + k2]
            s = ew * w + xlg[:, k2:k2 + 1] + xrt[k2:k2 + 1, :]
            t = c * jnp.abs(s)
            contrib = t if contrib is None else contrib + t
        acc_ref[:] += contrib
        return carry

    jax.lax.fori_loop(0, K1, kgroup, 0, unroll=False)

    mask = ew > thr_ref[0, 0]
    lm = jnp.where(mask, acc_ref[:], -1e30)
    mx = jnp.max(lm, axis=0, keepdims=True)
    ex = jnp.exp(lm - mx)
    den = jnp.sum(ex, axis=0, keepdims=True)
    rden = jnp.where(mx > -1e29, 1.0 / jnp.maximum(den, 1e-16), 0.0)
    alpha = ex * rden
    o = jax.lax.dot_general(alpha, xl_ref[:], (((0,), (0,)), ((), ())),
                            preferred_element_type=jnp.float32)
    out_ref[:] = jnp.maximum(o + b_ref[:], 0.0)


def _layer(thr, aw, att_row, we_row, ew, xlg, xrt, xl, al, art, b_row):
    smem = pl.BlockSpec(memory_space=pltpu.SMEM)
    return pl.pallas_call(
        _layer_body,
        grid=(N // BJ,),
        in_specs=[
            smem,
            smem,
            smem,
            smem,
            pl.BlockSpec((N, BJ), lambda j: (0, j)),
            pl.BlockSpec((K1, N, K2), lambda j: (0, 0, 0)),
            pl.BlockSpec((H, BJ), lambda j: (0, j)),
            pl.BlockSpec((N, H), lambda j: (0, 0)),
            pl.BlockSpec((N, 1), lambda j: (0, 0)),
            pl.BlockSpec((1, BJ), lambda j: (0, j)),
            pl.BlockSpec((1, H), lambda j: (0, 0)),
        ],
        out_specs=pl.BlockSpec((BJ, H), lambda j: (j, 0)),
        out_shape=jax.ShapeDtypeStruct((N, H), jnp.float32),
        scratch_shapes=[pltpu.VMEM((N, BJ), jnp.float32)],
    )(thr, aw, att_row, we_row, ew, xlg, xrt, xl, al, art, b_row)


def _head_body(x3_ref, w1_ref, b1_ref, w2_ref, b2_ref, out_ref):
    x4 = jnp.max(x3_ref[:], axis=0, keepdims=True)
    h = jnp.dot(x4.astype(jnp.bfloat16), w1_ref[:].astype(jnp.bfloat16),
                preferred_element_type=jnp.float32) + b1_ref[:]
    h = jnp.maximum(h, 0.0)
    out_ref[:] = jnp.dot(h.astype(jnp.bfloat16), w2_ref[:].astype(jnp.bfloat16),
                         preferred_element_type=jnp.float32) + b2_ref[:]


def _head(x3, w1, b1_row, w2, b2_row):
    return pl.pallas_call(
        _head_body,
        out_shape=jax.ShapeDtypeStruct((1, 1), jnp.float32),
    )(x3, w1, b1_row, w2, b2_row)


def kernel(features, edge_weights, threashold,
           W1_l, W1_r, We1, att1, b1,
           W2_l, W2_r, We2, att2, b2,
           W3_l, W3_r, We3, att3, b3,
           l1_W, l1_b, l2_W, l2_b):
    thr = (1.0 / jnp.asarray(threashold, jnp.float32)).reshape(1, 1)
    ew = edge_weights
    x = features
    for (wl, wr, we, att, b) in (
            (W1_l, W1_r, We1, att1, b1),
            (W2_l, W2_r, We2, att2, b2),
            (W3_l, W3_r, We3, att3, b3)):
        attc = att.reshape(H, 1)
        xl, xlg, xrt, al, art, aw = _prep(x, wl, wr, attc, we)
        x = _layer(thr, aw, att.reshape(1, H), we, ew,
                   xlg, xrt, xl, al, art, b.reshape(1, H))
    y = _head(x, l1_W, l1_b.reshape(1, 10), l2_W, l2_b.reshape(1, 1))
    return y.reshape(1)


# speedup vs baseline: 1.0982x; 1.0982x over previous
import jax
import jax.numpy as jnp
from jax.experimental import pallas as pl
from jax.experimental.pallas import tpu as pltpu

N = 1024
H = 64
BJ = 512
K1 = 8
K2 = 8

_HI = jax.lax.Precision.HIGHEST


def _layer_body(thr_ref, att_ref, we_ref,
                x_ref, wl_ref, wr_ref, attc_ref, wec_ref, ew_ref, b_ref,
                out_ref, xl_s, xlg_s, xrt3_s, al_s, acc_ref):
    j = pl.program_id(0)

    @pl.when(j == 0)
    def _prep():
        xb = x_ref[:].astype(jnp.bfloat16)
        xl = jnp.dot(xb, wl_ref[:].astype(jnp.bfloat16),
                     preferred_element_type=jnp.float32)
        xl_s[:] = xl
        for k1 in range(K1):
            xlg_s[k1] = xl[:, k1 * K2:(k1 + 1) * K2]
        xrt = jax.lax.dot_general(wr_ref[:].astype(jnp.bfloat16), xb,
                                  (((0,), (1,)), ((), ())),
                                  preferred_element_type=jnp.float32)
        for jj in range(N // BJ):
            xrt3_s[jj] = xrt[:, jj * BJ:(jj + 1) * BJ]
        al_s[:] = jnp.dot(xl, attc_ref[:], precision=_HI,
                          preferred_element_type=jnp.float32)

    ew = ew_ref[:]
    aw = jnp.dot(wec_ref[:], attc_ref[:], precision=_HI,
                 preferred_element_type=jnp.float32)
    acc_ref[:] = 0.6 * (al_s[:] + aw * ew)

    def kgroup(k1, carry):
        xlg = xlg_s[k1]
        xrt = xrt3_s[j, pl.ds(k1 * K2, K2), :]
        contrib = None
        for k2 in range(K2):
            w = we_ref[0, k1 * K2 + k2]
            c = 0.4 * att_ref[0, k1 * K2 + k2]
            s = ew * w + xlg[:, k2:k2 + 1] + xrt[k2:k2 + 1, :]
            t = c * jnp.abs(s)
            contrib = t if contrib is None else contrib + t
        acc_ref[:] += contrib
        return carry

    jax.lax.fori_loop(0, K1, kgroup, 0, unroll=False)

    mask = ew > thr_ref[0, 0]
    lm = jnp.where(mask, acc_ref[:], -1e30)
    mx = jnp.max(lm, axis=0, keepdims=True)
    ex = jnp.exp(lm - mx)
    den = jnp.sum(ex, axis=0, keepdims=True)
    rden = jnp.where(mx > -1e29, 1.0 / jnp.maximum(den, 1e-16), 0.0)
    alpha = ex * rden
    o = jax.lax.dot_general(alpha, xl_s[:], (((0,), (0,)), ((), ())),
                            preferred_element_type=jnp.float32)
    out_ref[:] = jnp.maximum(o + b_ref[:], 0.0)


def _layer(thr, att_row, we_row, x, wl, wr, attc, wec, ew, b_row):
    smem = pl.BlockSpec(memory_space=pltpu.SMEM)
    d = x.shape[1]
    return pl.pallas_call(
        _layer_body,
        grid=(N // BJ,),
        in_specs=[
            smem,
            smem,
            smem,
            pl.BlockSpec((N, d), lambda j: (0, 0)),
            pl.BlockSpec((d, H), lambda j: (0, 0)),
            pl.BlockSpec((d, H), lambda j: (0, 0)),
            pl.BlockSpec((H, 1), lambda j: (0, 0)),
            pl.BlockSpec((1, H), lambda j: (0, 0)),
            pl.BlockSpec((N, BJ), lambda j: (0, j)),
            pl.BlockSpec((1, H), lambda j: (0, 0)),
        ],
        out_specs=pl.BlockSpec((BJ, H), lambda j: (j, 0)),
        out_shape=jax.ShapeDtypeStruct((N, H), jnp.float32),
        scratch_shapes=[
            pltpu.VMEM((N, H), jnp.float32),
            pltpu.VMEM((K1, N, K2), jnp.float32),
            pltpu.VMEM((N // BJ, H, BJ), jnp.float32),
            pltpu.VMEM((N, 1), jnp.float32),
            pltpu.VMEM((N, BJ), jnp.float32),
        ],
    )(thr, att_row, we_row, x, wl, wr, attc, wec, ew, b_row)


def _head_body(x3_ref, w1_ref, b1_ref, w2_ref, b2_ref, out_ref):
    x4 = jnp.max(x3_ref[:], axis=0, keepdims=True)
    h = jnp.dot(x4.astype(jnp.bfloat16), w1_ref[:].astype(jnp.bfloat16),
                preferred_element_type=jnp.float32) + b1_ref[:]
    h = jnp.maximum(h, 0.0)
    out_ref[:] = jnp.dot(h.astype(jnp.bfloat16), w2_ref[:].astype(jnp.bfloat16),
                         preferred_element_type=jnp.float32) + b2_ref[:]


def _head(x3, w1, b1_row, w2, b2_row):
    return pl.pallas_call(
        _head_body,
        out_shape=jax.ShapeDtypeStruct((1, 1), jnp.float32),
    )(x3, w1, b1_row, w2, b2_row)


def kernel(features, edge_weights, threashold,
           W1_l, W1_r, We1, att1, b1,
           W2_l, W2_r, We2, att2, b2,
           W3_l, W3_r, We3, att3, b3,
           l1_W, l1_b, l2_W, l2_b):
    thr = (1.0 / jnp.asarray(threashold, jnp.float32)).reshape(1, 1)
    ew = edge_weights
    x = features
    for (wl, wr, we, att, b) in (
            (W1_l, W1_r, We1, att1, b1),
            (W2_l, W2_r, We2, att2, b2),
            (W3_l, W3_r, We3, att3, b3)):
        x = _layer(thr, att.reshape(1, H), we, x, wl, wr,
                   att.reshape(H, 1), we, ew, b.reshape(1, H))
    y = _head(x, l1_W, l1_b.reshape(1, 10), l2_W, l2_b.reshape(1, 1))
    return y.reshape(1)
